# Initial kernel scaffold; baseline (speedup 1.0000x reference)
import jax
import jax.numpy as jnp
from jax.experimental import pallas as pl


def kernel(x, edge_index, batch, W1l, b1, W1r, gamma1, beta1, W2l, b2, W2r, gamma2, beta2):
    raise NotImplementedError("write your pallas kernel here")


# speedup vs baseline: 5.1180x; 5.1180x over previous
import functools

import jax
import jax.numpy as jnp
from jax import lax
from jax.experimental import pallas as pl
from jax.experimental.pallas import tpu as pltpu
from jax.experimental.pallas import tpu_sc as plsc

N = 10000
E = 320000
D = 128
G = 64

NC = 2
NS = 16
DH = D // NC

C = 128
NCH = 160
EPAD = NS * NCH * C
CW = 8
ROWS_PT = 632
NPAD = NS * ROWS_PT


def _make_sc_agg(with_counts):
  mesh = plsc.VectorSubcoreMesh(core_axis_name="c", subcore_axis_name="s")
  s_type = jax.ShapeDtypeStruct((NC, NPAD, DH), jnp.float32)
  if with_counts:
    out_type = [s_type, jax.ShapeDtypeStruct((NPAD, CW), jnp.float32)]
  else:
    out_type = s_type

  @functools.partial(
      pl.kernel,
      out_type=out_type,
      mesh=mesh,
      compiler_params=pltpu.CompilerParams(use_tc_tiling_on_sc=False),
      scratch_types=[
          pltpu.VMEM((NCH, C), jnp.int32),
          pltpu.VMEM((NCH, C), jnp.int32),
          pltpu.VMEM((C, DH), jnp.float32),
          pltpu.VMEM((C, DH), jnp.float32),
          pltpu.VMEM((C, CW), jnp.float32),
          pltpu.VMEM_SHARED((NPAD, DH), jnp.float32),
          pltpu.VMEM_SHARED((NPAD, CW), jnp.float32),
          pltpu.SemaphoreType.DMA,
          pltpu.SemaphoreType.DMA,
      ],
  )
  def sc_agg(h0_hbm, h1_hbm, srcp_hbm, dstp_hbm, zrow_hbm, zcnt_hbm, ones_hbm,
             s_out, *rest):
    if with_counts:
      cnt_out = rest[0]
      scratch = rest[1:]
    else:
      cnt_out = None
      scratch = rest
    idx_s, idx_d, rows0, rows1, ones_v, sh_s, sh_cnt, sem0, sem1 = scratch

    cid = lax.axis_index("c")
    sid = lax.axis_index("s")
    rbase = sid * ROWS_PT

    pltpu.sync_copy(zrow_hbm, sh_s.at[pl.ds(rbase, ROWS_PT)])
    if with_counts:
      @pl.when(cid == 0)
      def _():
        pltpu.sync_copy(zcnt_hbm, sh_cnt.at[pl.ds(rbase, ROWS_PT)])
        pltpu.sync_copy(ones_hbm, ones_v)

    pltpu.sync_copy(srcp_hbm.at[sid], idx_s)
    pltpu.sync_copy(dstp_hbm.at[sid], idx_d)

    plsc.subcore_barrier()

    def run(table, do_counts):
      def gstart(j, buf, sem):
        pltpu.make_async_copy(table.at[idx_s.at[j]], buf, sem).start()

      def chunk(j, buf, sem):
        pltpu.make_async_copy(table.at[idx_s.at[j]], buf, sem).wait()
        pltpu.sync_copy(buf, sh_s.at[idx_d.at[j]], add=True)
        if do_counts:
          pltpu.sync_copy(ones_v, sh_cnt.at[idx_d.at[j]], add=True)

        @pl.when(j + 2 < NCH)
        def _():
          gstart(j + 2, buf, sem)

      gstart(0, rows0, sem0)
      gstart(1, rows1, sem1)

      def body(i, carry):
        chunk(2 * i, rows0, sem0)
        chunk(2 * i + 1, rows1, sem1)
        return carry

      lax.fori_loop(0, NCH // 2, body, 0)

    @pl.when(cid == 0)
    def _():
      run(h0_hbm, with_counts)

    @pl.when(cid == 1)
    def _():
      run(h1_hbm, False)

    plsc.subcore_barrier()

    pltpu.sync_copy(sh_s.at[pl.ds(rbase, ROWS_PT)],
                    s_out.at[cid, pl.ds(rbase, ROWS_PT)])
    if with_counts:
      @pl.when(cid == 0)
      def _():
        pltpu.sync_copy(sh_cnt.at[pl.ds(rbase, ROWS_PT)],
                        cnt_out.at[pl.ds(rbase, ROWS_PT)])

  return sc_agg


_sc_agg_counts = _make_sc_agg(True)
_sc_agg_plain = _make_sc_agg(False)


def _sage_bn_relu(s0, s1, cnt, h, Wl, b, Wr, g, be):
  agg = jnp.concatenate([s0, s1], axis=1) / jnp.maximum(cnt, 1.0)
  lin = lax.dot_general(agg, Wl, (((1,), (1,)), ((), ())),
                        preferred_element_type=jnp.float32)
  lin = lin + lax.dot_general(h, Wr, (((1,), (1,)), ((), ())),
                              preferred_element_type=jnp.float32) + b
  mu = jnp.mean(lin, axis=0, keepdims=True)
  var = jnp.mean((lin - mu) ** 2, axis=0, keepdims=True)
  xn = (lin - mu) * lax.rsqrt(var + 1e-5)
  return jnp.maximum(g * xn + be, 0.0)


def _tc_layer1_body(s0, s1, cnt, h, Wl, b, Wr, g, be, o):
  o[...] = _sage_bn_relu(s0[...], s1[...], cnt[...], h[...],
                         Wl[...], b[...], Wr[...], g[...], be[...])


def _tc_layer2_body(s0, s1, cnt, x1, Wl, b, Wr, g, be, batch2, o):
  x2 = _sage_bn_relu(s0[...], s1[...], cnt[...], x1[...],
                     Wl[...], b[...], Wr[...], g[...], be[...])
  hh = x1[...] + x2
  gid = lax.broadcasted_iota(jnp.int32, (N, G), 1)
  mask = (batch2[...] == gid).astype(jnp.float32)
  psum = lax.dot_general(mask, hh, (((0,), (0,)), ((), ())),
                         preferred_element_type=jnp.float32)
  cntg = jnp.sum(mask, axis=0)[:, None]
  o[...] = psum / jnp.maximum(cntg, 1.0)


_tc_layer1 = pl.pallas_call(
    _tc_layer1_body, out_shape=jax.ShapeDtypeStruct((N, D), jnp.float32))
_tc_layer2 = pl.pallas_call(
    _tc_layer2_body, out_shape=jax.ShapeDtypeStruct((G, D), jnp.float32))


@jax.jit
def kernel(x, edge_index, batch, W1l, b1, W1r, gamma1, beta1,
           W2l, b2, W2r, gamma2, beta2):
  src = edge_index[0].astype(jnp.int32)
  dst = edge_index[1].astype(jnp.int32)
  pad = EPAD - E
  srcp = jnp.concatenate([src, jnp.zeros((pad,), jnp.int32)]).reshape(
      NS, NCH, C)
  dstp = jnp.concatenate([dst, jnp.full((pad,), N, jnp.int32)]).reshape(
      NS, NCH, C)
  zrow = jnp.zeros((ROWS_PT, DH), jnp.float32)
  zcnt = jnp.zeros((ROWS_PT, CW), jnp.float32)
  ones = jnp.ones((C, CW), jnp.float32)
  batch2 = batch.astype(jnp.int32)[:, None]

  s1p, c1p = _sc_agg_counts(x[:, :DH], x[:, DH:], srcp, dstp,
                            zrow, zcnt, ones)
  cnt = c1p[:N, :1]
  x1 = _tc_layer1(s1p[0, :N], s1p[1, :N], cnt, x,
                  W1l, b1, W1r, gamma1, beta1)
  s2p = _sc_agg_plain(x1[:, :DH], x1[:, DH:], srcp, dstp,
                      zrow, zcnt, ones)
  return _tc_layer2(s2p[0, :N], s2p[1, :N], cnt, x1,
                    W2l, b2, W2r, gamma2, beta2, batch2)


# speedup vs baseline: 5.2954x; 1.0347x over previous
import functools

import jax
import jax.numpy as jnp
from jax import lax
from jax.experimental import pallas as pl
from jax.experimental.pallas import tpu as pltpu
from jax.experimental.pallas import tpu_sc as plsc

N = 10000
E = 320000
D = 128
G = 64

NC = 2
NS = 16
DH = D // NC

C = 128
NCH = 160
EPAD = NS * NCH * C
CW = 8
ROWS_PT = 632
NPAD = NS * ROWS_PT


def _make_sc_agg(with_counts):
  mesh = plsc.VectorSubcoreMesh(core_axis_name="c", subcore_axis_name="s")
  s_type = jax.ShapeDtypeStruct((NC, NPAD, DH), jnp.float32)
  if with_counts:
    out_type = [s_type, jax.ShapeDtypeStruct((NPAD, CW), jnp.float32)]
  else:
    out_type = s_type

  @functools.partial(
      pl.kernel,
      out_type=out_type,
      mesh=mesh,
      compiler_params=pltpu.CompilerParams(use_tc_tiling_on_sc=False),
      scratch_types=[
          pltpu.VMEM((NCH, C), jnp.int32),
          pltpu.VMEM((NCH, C), jnp.int32),
          pltpu.VMEM((C, DH), jnp.float32),
          pltpu.VMEM((C, DH), jnp.float32),
          pltpu.VMEM((C, DH), jnp.float32),
          pltpu.VMEM((C, DH), jnp.float32),
          pltpu.VMEM((C, CW), jnp.float32),
          pltpu.VMEM_SHARED((NPAD, DH), jnp.float32),
          pltpu.VMEM_SHARED((NPAD, CW), jnp.float32),
          pltpu.SemaphoreType.DMA,
          pltpu.SemaphoreType.DMA,
          pltpu.SemaphoreType.DMA,
          pltpu.SemaphoreType.DMA,
      ],
  )
  def sc_agg(h0_hbm, h1_hbm, srcp_hbm, dstp_hbm, zrow_hbm, zcnt_hbm, ones_hbm,
             s_out, *rest):
    if with_counts:
      cnt_out = rest[0]
      scratch = rest[1:]
    else:
      cnt_out = None
      scratch = rest
    (idx_s, idx_d, rows0, rows1, rows2, rows3, ones_v, sh_s, sh_cnt,
     sem0, sem1, sem2, sem3) = scratch
    bufs = (rows0, rows1, rows2, rows3)
    sems = (sem0, sem1, sem2, sem3)
    NB = 4

    cid = lax.axis_index("c")
    sid = lax.axis_index("s")
    rbase = sid * ROWS_PT

    pltpu.sync_copy(zrow_hbm, sh_s.at[pl.ds(rbase, ROWS_PT)])
    if with_counts:
      @pl.when(cid == 0)
      def _():
        pltpu.sync_copy(zcnt_hbm, sh_cnt.at[pl.ds(rbase, ROWS_PT)])
        pltpu.sync_copy(ones_hbm, ones_v)

    pltpu.sync_copy(srcp_hbm.at[sid], idx_s)
    pltpu.sync_copy(dstp_hbm.at[sid], idx_d)

    plsc.subcore_barrier()

    def run(table, do_counts):
      def gstart(j, buf, sem):
        pltpu.make_async_copy(table.at[idx_s.at[j]], buf, sem).start()

      def chunk(j, buf, sem):
        pltpu.make_async_copy(table.at[idx_s.at[j]], buf, sem).wait()
        pltpu.sync_copy(buf, sh_s.at[idx_d.at[j]], add=True)
        if do_counts:
          pltpu.sync_copy(ones_v, sh_cnt.at[idx_d.at[j]], add=True)

        @pl.when(j + NB < NCH)
        def _():
          gstart(j + NB, buf, sem)

      for b in range(NB):
        gstart(b, bufs[b], sems[b])

      def body(i, carry):
        for b in range(NB):
          chunk(NB * i + b, bufs[b], sems[b])
        return carry

      lax.fori_loop(0, NCH // NB, body, 0)

    @pl.when(cid == 0)
    def _():
      run(h0_hbm, with_counts)

    @pl.when(cid == 1)
    def _():
      run(h1_hbm, False)

    plsc.subcore_barrier()

    pltpu.sync_copy(sh_s.at[pl.ds(rbase, ROWS_PT)],
                    s_out.at[cid, pl.ds(rbase, ROWS_PT)])
    if with_counts:
      @pl.when(cid == 0)
      def _():
        pltpu.sync_copy(sh_cnt.at[pl.ds(rbase, ROWS_PT)],
                        cnt_out.at[pl.ds(rbase, ROWS_PT)])

  return sc_agg


_sc_agg_counts = _make_sc_agg(True)
_sc_agg_plain = _make_sc_agg(False)


def _sage_bn_relu(s0, s1, cnt, h, Wl, b, Wr, g, be):
  agg = jnp.concatenate([s0, s1], axis=1) / jnp.maximum(cnt, 1.0)
  lin = lax.dot_general(agg, Wl, (((1,), (1,)), ((), ())),
                        preferred_element_type=jnp.float32)
  lin = lin + lax.dot_general(h, Wr, (((1,), (1,)), ((), ())),
                              preferred_element_type=jnp.float32) + b
  mu = jnp.mean(lin, axis=0, keepdims=True)
  var = jnp.mean((lin - mu) ** 2, axis=0, keepdims=True)
  xn = (lin - mu) * lax.rsqrt(var + 1e-5)
  return jnp.maximum(g * xn + be, 0.0)


def _tc_layer1_body(s0, s1, cnt, h, Wl, b, Wr, g, be, o):
  o[...] = _sage_bn_relu(s0[...], s1[...], cnt[...], h[...],
                         Wl[...], b[...], Wr[...], g[...], be[...])


def _tc_layer2_body(s0, s1, cnt, x1, Wl, b, Wr, g, be, batch2, o):
  x2 = _sage_bn_relu(s0[...], s1[...], cnt[...], x1[...],
                     Wl[...], b[...], Wr[...], g[...], be[...])
  hh = x1[...] + x2
  gid = lax.broadcasted_iota(jnp.int32, (N, G), 1)
  mask = (batch2[...] == gid).astype(jnp.float32)
  psum = lax.dot_general(mask, hh, (((0,), (0,)), ((), ())),
                         preferred_element_type=jnp.float32)
  cntg = jnp.sum(mask, axis=0)[:, None]
  o[...] = psum / jnp.maximum(cntg, 1.0)


_tc_layer1 = pl.pallas_call(
    _tc_layer1_body, out_shape=jax.ShapeDtypeStruct((N, D), jnp.float32))
_tc_layer2 = pl.pallas_call(
    _tc_layer2_body, out_shape=jax.ShapeDtypeStruct((G, D), jnp.float32))


@jax.jit
def kernel(x, edge_index, batch, W1l, b1, W1r, gamma1, beta1,
           W2l, b2, W2r, gamma2, beta2):
  src = edge_index[0].astype(jnp.int32)
  dst = edge_index[1].astype(jnp.int32)
  pad = EPAD - E
  srcp = jnp.concatenate([src, jnp.zeros((pad,), jnp.int32)]).reshape(
      NS, NCH, C)
  dstp = jnp.concatenate([dst, jnp.full((pad,), N, jnp.int32)]).reshape(
      NS, NCH, C)
  zrow = jnp.zeros((ROWS_PT, DH), jnp.float32)
  zcnt = jnp.zeros((ROWS_PT, CW), jnp.float32)
  ones = jnp.ones((C, CW), jnp.float32)
  batch2 = batch.astype(jnp.int32)[:, None]

  s1p, c1p = _sc_agg_counts(x[:, :DH], x[:, DH:], srcp, dstp,
                            zrow, zcnt, ones)
  cnt = c1p[:N, :1]
  x1 = _tc_layer1(s1p[0, :N], s1p[1, :N], cnt, x,
                  W1l, b1, W1r, gamma1, beta1)
  s2p = _sc_agg_plain(x1[:, :DH], x1[:, DH:], srcp, dstp,
                      zrow, zcnt, ones)
  return _tc_layer2(s2p[0, :N], s2p[1, :N], cnt, x1,
                    W2l, b2, W2r, gamma2, beta2, batch2)
